# baseline (speedup 1.0000x reference)
import functools

import jax
import jax.numpy as jnp
from jax import lax
from jax.experimental import pallas as pl
from jax.experimental.pallas import tpu as pltpu
from jax.experimental.pallas import tpu_sc as plsc

B, N, FD, G = 4, 2048, 64, 64
H = 128
BN = B * N
NC, NS, L = 2, 16, 16
NW = NC * NS
PTS = BN // NW
TPB = NW // B
ROWS = G // TPB
FV = FD // L
NEG = -3.0e38


def _tc_body(xT_ref, sph_ref, shp_ref, w1_ref, b1_ref, g1_ref, be1_ref,
             w2_ref, b2_ref, g2_ref, be2_ref,
             soft_ref, wxyz_ref, wfold_ref, grp_ref):
    xT = xT_ref[...]
    h = jnp.dot(w1_ref[...], xT, preferred_element_type=jnp.float32)
    h = h + b1_ref[...]
    mean = jnp.mean(h, axis=1, keepdims=True)
    var = jnp.mean((h - mean) ** 2, axis=1, keepdims=True)
    h = g1_ref[...] * (h - mean) * lax.rsqrt(var + 1e-5) + be1_ref[...]
    h = jnp.maximum(h, 0.0)
    s = jnp.dot(w2_ref[...], h, preferred_element_type=jnp.float32)
    s = s + b2_ref[...]
    mean2 = jnp.mean(s, axis=1, keepdims=True)
    var2 = jnp.mean((s - mean2) ** 2, axis=1, keepdims=True)
    s = g2_ref[...] * (s - mean2) * lax.rsqrt(var2 + 1e-5) + be2_ref[...]
    for b in range(B):
        sb = s[:, b * N:(b + 1) * N]
        m = jnp.max(sb, axis=1, keepdims=True)
        e = jnp.exp(sb - m)
        soft_b = e / jnp.sum(e, axis=1, keepdims=True)
        soft_ref[b] = soft_b
        wxyz_ref[b] = jnp.dot(soft_b, sph_ref[b],
                              preferred_element_type=jnp.float32)
        wfold_ref[b] = jnp.dot(soft_b, shp_ref[b],
                               preferred_element_type=jnp.float32)
        mcol = jnp.max(sb, axis=0, keepdims=True)
        ids = lax.broadcasted_iota(jnp.int32, (G, N), 0)
        cand = jnp.where(sb == mcol, ids, G)
        grp_ref[b] = jnp.min(cand, axis=0)


_tc_call = pl.pallas_call(
    _tc_body,
    out_shape=(
        jax.ShapeDtypeStruct((B, G, N), jnp.float32),
        jax.ShapeDtypeStruct((B, G, 3), jnp.float32),
        jax.ShapeDtypeStruct((B, G, 3), jnp.float32),
        jax.ShapeDtypeStruct((B, N), jnp.int32),
    ),
)


def _sc_body(grp_hbm, feat_hbm, gf_hbm, sf_hbm,
             grp_v, feat_v, tab_v, full_v, acc_v, src_v, facc_v, fsrc_v,
             out_v, hbm_tab, hbm_full):
    c = lax.axis_index("c")
    s = lax.axis_index("s")
    bl = s // TPB
    b = c * 2 + bl
    chunk = s % TPB
    pt0 = b * N + chunk * PTS
    wid = c * NS + s

    pltpu.sync_copy(grp_hbm.at[pl.ds(pt0, PTS)], grp_v)
    pltpu.sync_copy(feat_hbm.at[pl.ds(pt0, PTS), :], feat_v)

    for g in range(G):
        for j in range(FV):
            tab_v[g, pl.ds(j * L, L)] = jnp.full((L,), NEG, jnp.float32)

    def point_body(k, carry):
        gvec = grp_v[pl.ds(k * L, L)]
        for m in range(L):
            g = gvec[m]
            i = k * L + m
            for j in range(FV):
                col = pl.ds(j * L, L)
                tab_v[g, col] = jnp.maximum(tab_v[g, col], feat_v[i, col])
        return carry

    lax.fori_loop(0, PTS // L, point_body, 0)

    vmin = grp_v[pl.ds(0, L)]
    vmax = vmin
    for k in range(1, PTS // L):
        w = grp_v[pl.ds(k * L, L)]
        vmin = jnp.minimum(vmin, w)
        vmax = jnp.maximum(vmax, w)
    cmin = jnp.min(vmin)
    cmax = jnp.max(vmax)
    uniform = cmin == cmax
    for j in range(FV):
        ids = lax.iota(jnp.int32, L) + j * L
        flag = jnp.where((ids == cmin) & uniform, 1.0, 0.0)
        full_v[pl.ds(j * L, L)] = flag

    pltpu.sync_copy(tab_v, hbm_tab.at[wid])
    pltpu.sync_copy(full_v, hbm_full.at[wid])
    plsc.subcore_barrier()

    w0 = c * NS + bl * TPB
    r = s % TPB
    pltpu.sync_copy(hbm_tab.at[w0, pl.ds(r * ROWS, ROWS), :], acc_v)
    pltpu.sync_copy(hbm_full.at[w0], facc_v.at[pl.ds(0, G)])
    for t in range(1, TPB):
        pltpu.sync_copy(hbm_tab.at[w0 + t, pl.ds(r * ROWS, ROWS), :], src_v)
        pltpu.sync_copy(hbm_full.at[w0 + t], fsrc_v)
        for rr in range(ROWS):
            for j in range(FV):
                col = pl.ds(j * L, L)
                acc_v[rr, col] = jnp.maximum(acc_v[rr, col], src_v[rr, col])
        for j in range(FV):
            col = pl.ds(j * L, L)
            facc_v[col] = jnp.minimum(facc_v[col], fsrc_v[col])

    fvec = facc_v[pl.ds(r * ROWS, L)]
    for rr in range(ROWS):
        fb = fvec[rr]
        floor = jnp.where(fb > 0.5, NEG, 0.0).astype(jnp.float32)
        for j in range(FV):
            col = pl.ds(j * L, L)
            acc_v[rr, col] = jnp.maximum(acc_v[rr, col], floor)

    pltpu.sync_copy(acc_v, gf_hbm.at[pl.ds(b * G + r * ROWS, ROWS), :])
    plsc.subcore_barrier()

    pltpu.sync_copy(gf_hbm.at[pl.ds(b * G, G), :], tab_v)

    def gather_body(k, carry):
        gvec = grp_v[pl.ds(k * L, L)]
        for m in range(L):
            g = gvec[m]
            i = k * L + m
            for j in range(FV):
                col = pl.ds(j * L, L)
                out_v[i, col] = tab_v[g, col]
        return carry

    lax.fori_loop(0, PTS // L, gather_body, 0)
    pltpu.sync_copy(out_v, sf_hbm.at[pl.ds(pt0, PTS), :])


@functools.cache
def _get_sc_call():
    return functools.partial(
        pl.kernel,
        out_type=(
            jax.ShapeDtypeStruct((B * G, FD), jnp.float32),
            jax.ShapeDtypeStruct((BN, FD), jnp.float32),
        ),
        mesh=plsc.VectorSubcoreMesh(core_axis_name="c", subcore_axis_name="s",
                                    num_cores=NC, num_subcores=NS),
        compiler_params=pltpu.CompilerParams(needs_layout_passes=False),
        scratch_types=[
            pltpu.VMEM((PTS,), jnp.int32),
            pltpu.VMEM((PTS, FD), jnp.float32),
            pltpu.VMEM((G, FD), jnp.float32),
            pltpu.VMEM((G,), jnp.float32),
            pltpu.VMEM((ROWS, FD), jnp.float32),
            pltpu.VMEM((ROWS, FD), jnp.float32),
            pltpu.VMEM((G + L,), jnp.float32),
            pltpu.VMEM((G,), jnp.float32),
            pltpu.VMEM((PTS, FD), jnp.float32),
            pltpu.HBM((NW, G, FD), jnp.float32),
            pltpu.HBM((NW, G), jnp.float32),
        ],
    )(_sc_body)


def kernel(sphere, shape, features, w1, b1, g1, be1, w2, b2, g2, be2):
    xT = sphere.reshape(BN, 3).T
    col = lambda v: v.reshape(-1, 1)
    soft, wxyz, wfold, groups = _tc_call(
        xT, sphere, shape, w1, col(b1), col(g1), col(be1),
        w2, col(b2), col(g2), col(be2))
    gf, sf = _get_sc_call()(groups.reshape(BN), features.reshape(BN, FD))
    return (soft, wxyz, groups, gf.reshape(B, G, FD),
            sf.reshape(B, N, FD), wfold)


# speedup vs baseline: 1.1406x; 1.1406x over previous
import functools

import jax
import jax.numpy as jnp
from jax import lax
from jax.experimental import pallas as pl
from jax.experimental.pallas import tpu as pltpu
from jax.experimental.pallas import tpu_sc as plsc

B, N, FD, G = 4, 2048, 64, 64
H = 128
BN = B * N
NC, NS, L = 2, 16, 16
NW = NC * NS
PTS = BN // NW
TPB = NW // B
ROWS = G // TPB
FV = FD // L
NEG = -3.0e38


def _tc_body(xT_ref, sph_ref, shp_ref, w1_ref, b1_ref, g1_ref, be1_ref,
             w2_ref, b2_ref, g2_ref, be2_ref,
             soft_ref, wxyz_ref, wfold_ref, grp_ref):
    xT = xT_ref[...]
    h = jnp.dot(w1_ref[...], xT, preferred_element_type=jnp.float32)
    h = h + b1_ref[...]
    mean = jnp.mean(h, axis=1, keepdims=True)
    var = jnp.mean((h - mean) ** 2, axis=1, keepdims=True)
    h = g1_ref[...] * (h - mean) * lax.rsqrt(var + 1e-5) + be1_ref[...]
    h = jnp.maximum(h, 0.0)
    s = jnp.dot(w2_ref[...], h, preferred_element_type=jnp.float32)
    s = s + b2_ref[...]
    mean2 = jnp.mean(s, axis=1, keepdims=True)
    var2 = jnp.mean((s - mean2) ** 2, axis=1, keepdims=True)
    s = g2_ref[...] * (s - mean2) * lax.rsqrt(var2 + 1e-5) + be2_ref[...]
    for b in range(B):
        sb = s[:, b * N:(b + 1) * N]
        m = jnp.max(sb, axis=1, keepdims=True)
        e = jnp.exp(sb - m)
        soft_b = e / jnp.sum(e, axis=1, keepdims=True)
        soft_ref[b] = soft_b
        wxyz_ref[b] = jnp.dot(soft_b, sph_ref[b],
                              preferred_element_type=jnp.float32)
        wfold_ref[b] = jnp.dot(soft_b, shp_ref[b],
                               preferred_element_type=jnp.float32)
        mcol = jnp.max(sb, axis=0, keepdims=True)
        ids = lax.broadcasted_iota(jnp.int32, (G, N), 0)
        cand = jnp.where(sb == mcol, ids, G)
        grp_ref[b] = jnp.min(cand, axis=0)


_tc_call = pl.pallas_call(
    _tc_body,
    out_shape=(
        jax.ShapeDtypeStruct((B, G, N), jnp.float32),
        jax.ShapeDtypeStruct((B, G, 3), jnp.float32),
        jax.ShapeDtypeStruct((B, G, 3), jnp.float32),
        jax.ShapeDtypeStruct((B, N), jnp.int32),
    ),
)


def _tc2_body(grp_ref, gf_ref, sf_ref):
    for b in range(B):
        gid = grp_ref[b]
        oh = (lax.broadcasted_iota(jnp.int32, (N, G), 1) == gid)
        sf_ref[b] = jnp.dot(oh.astype(jnp.float32),
                            gf_ref[b * G:(b + 1) * G, :],
                            preferred_element_type=jnp.float32)


_tc2_call = pl.pallas_call(
    _tc2_body,
    out_shape=jax.ShapeDtypeStruct((B, N, FD), jnp.float32),
)


def _sc_body(grp_hbm, feat_hbm, gf_hbm,
             grp_v, feat_v, tab_v, full_v, red_v, fblk_v, facc_v, acc_v,
             sem, hbm_tab, hbm_full):
    c = lax.axis_index("c")
    s = lax.axis_index("s")
    bl = s // TPB
    b = c * 2 + bl
    chunk = s % TPB
    pt0 = b * N + chunk * PTS
    wid = c * NS + s

    pltpu.sync_copy(grp_hbm.at[pl.ds(pt0, PTS)], grp_v)
    pltpu.sync_copy(feat_hbm.at[pl.ds(pt0, PTS), :], feat_v)

    for g in range(G):
        for j in range(FV):
            tab_v[g, pl.ds(j * L, L)] = jnp.full((L,), NEG, jnp.float32)

    def point_body(k, carry):
        gvec = grp_v[pl.ds(k * L, L)]
        for m in range(L):
            g = gvec[m]
            i = k * L + m
            for j in range(FV):
                col = pl.ds(j * L, L)
                tab_v[g, col] = jnp.maximum(tab_v[g, col], feat_v[i, col])
        return carry

    lax.fori_loop(0, PTS // L, point_body, 0)

    vmin = grp_v[pl.ds(0, L)]
    vmax = vmin
    for k in range(1, PTS // L):
        w = grp_v[pl.ds(k * L, L)]
        vmin = jnp.minimum(vmin, w)
        vmax = jnp.maximum(vmax, w)
    cmin = jnp.min(vmin)
    cmax = jnp.max(vmax)
    uniform = cmin == cmax
    for j in range(FV):
        ids = lax.iota(jnp.int32, L) + j * L
        flag = jnp.where((ids == cmin) & uniform, 1.0, 0.0)
        full_v[pl.ds(j * L, L)] = flag

    pltpu.sync_copy(tab_v, hbm_tab.at[wid])
    pltpu.sync_copy(full_v, hbm_full.at[wid])
    plsc.subcore_barrier()

    w0 = c * NS + bl * TPB
    r = s % TPB
    cps = [pltpu.make_async_copy(
        hbm_tab.at[w0 + t, pl.ds(r * ROWS, ROWS), :], red_v.at[t], sem)
        for t in range(TPB)]
    cps.append(pltpu.make_async_copy(
        hbm_full.at[pl.ds(w0, TPB), :], fblk_v, sem))
    for cp in cps:
        cp.start()
    for cp in cps:
        cp.wait()

    for j in range(FV):
        col = pl.ds(j * L, L)
        fmin = fblk_v[0, col]
        for t in range(1, TPB):
            fmin = jnp.minimum(fmin, fblk_v[t, col])
        facc_v[col] = fmin
    fvec = facc_v[pl.ds(r * ROWS, L)]

    for rr in range(ROWS):
        fb = fvec[rr]
        floor = jnp.where(fb > 0.5, NEG, 0.0).astype(jnp.float32)
        for j in range(FV):
            col = pl.ds(j * L, L)
            m = jnp.maximum(red_v[0, rr, col], red_v[1, rr, col])
            for t in range(2, TPB):
                m = jnp.maximum(m, red_v[t, rr, col])
            acc_v[rr, col] = jnp.maximum(m, floor)

    pltpu.sync_copy(acc_v, gf_hbm.at[pl.ds(b * G + r * ROWS, ROWS), :])


@functools.cache
def _get_sc_call():
    return functools.partial(
        pl.kernel,
        out_type=jax.ShapeDtypeStruct((B * G, FD), jnp.float32),
        mesh=plsc.VectorSubcoreMesh(core_axis_name="c", subcore_axis_name="s",
                                    num_cores=NC, num_subcores=NS),
        compiler_params=pltpu.CompilerParams(needs_layout_passes=False),
        scratch_types=[
            pltpu.VMEM((PTS,), jnp.int32),
            pltpu.VMEM((PTS, FD), jnp.float32),
            pltpu.VMEM((G, FD), jnp.float32),
            pltpu.VMEM((G,), jnp.float32),
            pltpu.VMEM((TPB, ROWS, FD), jnp.float32),
            pltpu.VMEM((TPB, G), jnp.float32),
            pltpu.VMEM((G + L,), jnp.float32),
            pltpu.VMEM((ROWS, FD), jnp.float32),
            pltpu.SemaphoreType.DMA,
            pltpu.HBM((NW, G, FD), jnp.float32),
            pltpu.HBM((NW, G), jnp.float32),
        ],
    )(_sc_body)


def kernel(sphere, shape, features, w1, b1, g1, be1, w2, b2, g2, be2):
    xT = sphere.reshape(BN, 3).T
    col = lambda v: v.reshape(-1, 1)
    soft, wxyz, wfold, groups = _tc_call(
        xT, sphere, shape, w1, col(b1), col(g1), col(be1),
        w2, col(b2), col(g2), col(be2))
    gf = _get_sc_call()(groups.reshape(BN), features.reshape(BN, FD))
    sf = _tc2_call(groups.reshape(B, N, 1), gf)
    return (soft, wxyz, groups, gf.reshape(B, G, FD), sf, wfold)


# speedup vs baseline: 1.2884x; 1.1295x over previous
import functools

import jax
import jax.numpy as jnp
from jax import lax
from jax.experimental import pallas as pl
from jax.experimental.pallas import tpu as pltpu
from jax.experimental.pallas import tpu_sc as plsc

B, N, FD, G = 4, 2048, 64, 64
H = 128
BN = B * N
NC, NS, L = 2, 16, 16
NW = NC * NS
PTS = BN // NW
TPB = NW // B
ROWS = G // TPB
FV = FD // L
NEG = -3.0e38


def _eye(n):
    return (lax.broadcasted_iota(jnp.int32, (n, n), 0) ==
            lax.broadcasted_iota(jnp.int32, (n, n), 1)).astype(jnp.float32)


def _col(v_ref, eye):
    return lax.dot_general(eye, v_ref[...][None, :], (((1,), (1,)), ((), ())),
                           preferred_element_type=jnp.float32)


def _tc_body(sph_ref, shp_ref, w1_ref, b1_ref, g1_ref, be1_ref,
             w2_ref, b2_ref, g2_ref, be2_ref,
             soft_ref, wxyz_ref, wfold_ref, grp_ref, grpl_ref):
    dn = (((1,), (1,)), ((), ()))
    eyeH, eyeG = _eye(H), _eye(G)
    x = sph_ref[...].reshape(BN, 3)
    xT = lax.dot_general(_eye(3), x, dn,
                         preferred_element_type=jnp.float32)
    h = jnp.dot(w1_ref[...], xT, preferred_element_type=jnp.float32)
    h = h + _col(b1_ref, eyeH)
    mean = jnp.mean(h, axis=1, keepdims=True)
    var = jnp.mean((h - mean) ** 2, axis=1, keepdims=True)
    h = (_col(g1_ref, eyeH) * (h - mean) * lax.rsqrt(var + 1e-5)
         + _col(be1_ref, eyeH))
    h = jnp.maximum(h, 0.0)
    s = jnp.dot(w2_ref[...], h, preferred_element_type=jnp.float32)
    s = s + _col(b2_ref, eyeG)
    mean2 = jnp.mean(s, axis=1, keepdims=True)
    var2 = jnp.mean((s - mean2) ** 2, axis=1, keepdims=True)
    s = (_col(g2_ref, eyeG) * (s - mean2) * lax.rsqrt(var2 + 1e-5)
         + _col(be2_ref, eyeG))
    for b in range(B):
        sb = s[:, b * N:(b + 1) * N]
        m = jnp.max(sb, axis=1, keepdims=True)
        e = jnp.exp(sb - m)
        soft_b = e / jnp.sum(e, axis=1, keepdims=True)
        soft_ref[b] = soft_b
        wxyz_ref[b] = jnp.dot(soft_b, sph_ref[b],
                              preferred_element_type=jnp.float32)
        wfold_ref[b] = jnp.dot(soft_b, shp_ref[b],
                               preferred_element_type=jnp.float32)
        mcol = jnp.max(sb, axis=0, keepdims=True)
        ids = lax.broadcasted_iota(jnp.int32, (G, N), 0)
        grp_b = jnp.min(jnp.where(sb == mcol, ids, G), axis=0)
        grp_ref[b] = grp_b
        grpl_ref[pl.ds(b * N, N)] = grp_b


_tc_call = pl.pallas_call(
    _tc_body,
    out_shape=(
        jax.ShapeDtypeStruct((B, G, N), jnp.float32),
        jax.ShapeDtypeStruct((B, G, 3), jnp.float32),
        jax.ShapeDtypeStruct((B, G, 3), jnp.float32),
        jax.ShapeDtypeStruct((B, N), jnp.int32),
        jax.ShapeDtypeStruct((BN,), jnp.int32),
    ),
)


def _tc2_body(grp_ref, gf_ref, sf_ref):
    for b in range(B):
        grp_row = grp_ref[b][None, :]
        ohT = (lax.broadcasted_iota(jnp.int32, (G, N), 0) == grp_row)
        sf_ref[b] = lax.dot_general(
            ohT.astype(jnp.float32), gf_ref[b * G:(b + 1) * G, :],
            (((0,), (0,)), ((), ())), preferred_element_type=jnp.float32)


_tc2_call = pl.pallas_call(
    _tc2_body,
    out_shape=jax.ShapeDtypeStruct((B, N, FD), jnp.float32),
)


def _sc_body(grp_hbm, feat_hbm, gf_hbm,
             grp_v, feat_v, tab_v, full_v, red_v, fblk_v, facc_v, acc_v,
             sem, hbm_tab, hbm_full):
    c = lax.axis_index("c")
    s = lax.axis_index("s")
    bl = s // TPB
    b = c * 2 + bl
    chunk = s % TPB
    pt0 = b * N + chunk * PTS
    wid = c * NS + s

    pltpu.sync_copy(grp_hbm.at[pl.ds(pt0, PTS)], grp_v)
    pltpu.sync_copy(feat_hbm.at[pl.ds(pt0, PTS), :], feat_v)

    for g in range(G):
        for j in range(FV):
            tab_v[g, pl.ds(j * L, L)] = jnp.full((L,), NEG, jnp.float32)

    def point_body(k, carry):
        gvec = grp_v[pl.ds(k * L, L)]
        for m in range(L):
            g = gvec[m]
            i = k * L + m
            for j in range(FV):
                col = pl.ds(j * L, L)
                tab_v[g, col] = jnp.maximum(tab_v[g, col], feat_v[i, col])
        return carry

    lax.fori_loop(0, PTS // L, point_body, 0)

    vmin = grp_v[pl.ds(0, L)]
    vmax = vmin
    for k in range(1, PTS // L):
        w = grp_v[pl.ds(k * L, L)]
        vmin = jnp.minimum(vmin, w)
        vmax = jnp.maximum(vmax, w)
    cmin = jnp.min(vmin)
    cmax = jnp.max(vmax)
    uniform = cmin == cmax
    for j in range(FV):
        ids = lax.iota(jnp.int32, L) + j * L
        flag = jnp.where((ids == cmin) & uniform, 1.0, 0.0)
        full_v[pl.ds(j * L, L)] = flag

    pltpu.sync_copy(tab_v, hbm_tab.at[wid])
    pltpu.sync_copy(full_v, hbm_full.at[wid])
    plsc.subcore_barrier()

    w0 = c * NS + bl * TPB
    r = s % TPB
    cps = [pltpu.make_async_copy(
        hbm_tab.at[w0 + t, pl.ds(r * ROWS, ROWS), :], red_v.at[t], sem)
        for t in range(TPB)]
    cps.append(pltpu.make_async_copy(
        hbm_full.at[pl.ds(w0, TPB), :], fblk_v, sem))
    for cp in cps:
        cp.start()
    for cp in cps:
        cp.wait()

    for j in range(FV):
        col = pl.ds(j * L, L)
        fmin = fblk_v[0, col]
        for t in range(1, TPB):
            fmin = jnp.minimum(fmin, fblk_v[t, col])
        facc_v[col] = fmin
    fvec = facc_v[pl.ds(r * ROWS, L)]

    for rr in range(ROWS):
        fb = fvec[rr]
        floor = jnp.where(fb > 0.5, NEG, 0.0).astype(jnp.float32)
        for j in range(FV):
            col = pl.ds(j * L, L)
            m = jnp.maximum(red_v[0, rr, col], red_v[1, rr, col])
            for t in range(2, TPB):
                m = jnp.maximum(m, red_v[t, rr, col])
            acc_v[rr, col] = jnp.maximum(m, floor)

    pltpu.sync_copy(acc_v, gf_hbm.at[pl.ds(b * G + r * ROWS, ROWS), :])


@functools.cache
def _get_sc_call():
    return functools.partial(
        pl.kernel,
        out_type=jax.ShapeDtypeStruct((B * G, FD), jnp.float32),
        mesh=plsc.VectorSubcoreMesh(core_axis_name="c", subcore_axis_name="s",
                                    num_cores=NC, num_subcores=NS),
        compiler_params=pltpu.CompilerParams(needs_layout_passes=False),
        scratch_types=[
            pltpu.VMEM((PTS,), jnp.int32),
            pltpu.VMEM((PTS, FD), jnp.float32),
            pltpu.VMEM((G, FD), jnp.float32),
            pltpu.VMEM((G,), jnp.float32),
            pltpu.VMEM((TPB, ROWS, FD), jnp.float32),
            pltpu.VMEM((TPB, G), jnp.float32),
            pltpu.VMEM((G + L,), jnp.float32),
            pltpu.VMEM((ROWS, FD), jnp.float32),
            pltpu.SemaphoreType.DMA,
            pltpu.HBM((NW, G, FD), jnp.float32),
            pltpu.HBM((NW, G), jnp.float32),
        ],
    )(_sc_body)


def kernel(sphere, shape, features, w1, b1, g1, be1, w2, b2, g2, be2):
    soft, wxyz, wfold, groups, grpl = _tc_call(
        sphere, shape, w1, b1, g1, be1, w2, b2, g2, be2)
    gf = _get_sc_call()(grpl, features.reshape(BN, FD))
    sf = _tc2_call(groups, gf)
    return (soft, wxyz, groups, gf.reshape(B, G, FD), sf, wfold)


# speedup vs baseline: 1.4280x; 1.1084x over previous
import functools

import jax
import jax.numpy as jnp
from jax import lax
from jax.experimental import pallas as pl
from jax.experimental.pallas import tpu as pltpu
from jax.experimental.pallas import tpu_sc as plsc

B, N, FD, G = 4, 2048, 64, 64
H = 128
BN = B * N
NC, NS, L = 2, 16, 16
NW = NC * NS
PTS = BN // NW
TPB = NW // B
ROWS = G // TPB
FV = FD // L
NEG = -3.0e38


def _eye(n):
    return (lax.broadcasted_iota(jnp.int32, (n, n), 0) ==
            lax.broadcasted_iota(jnp.int32, (n, n), 1)).astype(jnp.float32)


def _col(v_ref, eye):
    return lax.dot_general(eye, v_ref[...][None, :], (((1,), (1,)), ((), ())),
                           preferred_element_type=jnp.float32)


def _tc_body(xT_ref, sT_ref, w1_ref, b1_ref, g1_ref, be1_ref,
             w2_ref, b2_ref, g2_ref, be2_ref,
             soft_ref, wxyz_ref, wfold_ref, grp_ref, grpl_ref):
    dn = (((1,), (1,)), ((), ()))
    eyeH, eyeG = _eye(H), _eye(G)
    xT = xT_ref[...]
    h = jnp.dot(w1_ref[...], xT, preferred_element_type=jnp.float32)
    h = h + _col(b1_ref, eyeH)
    mean = jnp.mean(h, axis=1, keepdims=True)
    var = jnp.mean((h - mean) ** 2, axis=1, keepdims=True)
    h = (_col(g1_ref, eyeH) * (h - mean) * lax.rsqrt(var + 1e-5)
         + _col(be1_ref, eyeH))
    h = jnp.maximum(h, 0.0)
    s = jnp.dot(w2_ref[...], h, preferred_element_type=jnp.float32)
    s = s + _col(b2_ref, eyeG)
    mean2 = jnp.mean(s, axis=1, keepdims=True)
    var2 = jnp.mean((s - mean2) ** 2, axis=1, keepdims=True)
    s = (_col(g2_ref, eyeG) * (s - mean2) * lax.rsqrt(var2 + 1e-5)
         + _col(be2_ref, eyeG))
    for b in range(B):
        sb = s[:, b * N:(b + 1) * N]
        m = jnp.max(sb, axis=1, keepdims=True)
        e = jnp.exp(sb - m)
        soft_b = e / jnp.sum(e, axis=1, keepdims=True)
        soft_ref[b] = soft_b
        wxyz_ref[:, b, :] = lax.dot_general(
            xT[:, b * N:(b + 1) * N], soft_b, dn,
            preferred_element_type=jnp.float32)
        wfold_ref[:, b, :] = lax.dot_general(
            sT_ref[:, b * N:(b + 1) * N], soft_b, dn,
            preferred_element_type=jnp.float32)
        mcol = jnp.max(sb, axis=0, keepdims=True)
        ids = lax.broadcasted_iota(jnp.int32, (G, N), 0)
        grp_b = jnp.min(jnp.where(sb == mcol, ids, G), axis=0)
        grp_ref[b] = grp_b
        grpl_ref[pl.ds(b * N, N)] = grp_b


_tc_call = pl.pallas_call(
    _tc_body,
    out_shape=(
        jax.ShapeDtypeStruct((B, G, N), jnp.float32),
        jax.ShapeDtypeStruct((3, B, G), jnp.float32),
        jax.ShapeDtypeStruct((3, B, G), jnp.float32),
        jax.ShapeDtypeStruct((B, N), jnp.int32),
        jax.ShapeDtypeStruct((BN,), jnp.int32),
    ),
)


def _tc2_body(grp_ref, gf_ref, sf_ref):
    for b in range(B):
        grp_row = grp_ref[b][None, :]
        ohT = (lax.broadcasted_iota(jnp.int32, (G, N), 0) == grp_row)
        sf_ref[b] = lax.dot_general(
            ohT.astype(jnp.float32), gf_ref[b * G:(b + 1) * G, :],
            (((0,), (0,)), ((), ())), preferred_element_type=jnp.float32)


_tc2_call = pl.pallas_call(
    _tc2_body,
    out_shape=jax.ShapeDtypeStruct((B, N, FD), jnp.float32),
)


def _sc_body(grp_hbm, feat_hbm, gf_hbm,
             grp_v, feat_v, tab_v, full_v, red_v, fblk_v, facc_v, acc_v,
             sem, hbm_tab, hbm_full):
    c = lax.axis_index("c")
    s = lax.axis_index("s")
    bl = s // TPB
    b = c * 2 + bl
    chunk = s % TPB
    pt0 = b * N + chunk * PTS
    wid = c * NS + s

    pltpu.sync_copy(grp_hbm.at[pl.ds(pt0, PTS)], grp_v)
    pltpu.sync_copy(feat_hbm.at[b, pl.ds(chunk * PTS, PTS), :], feat_v)

    for g in range(G):
        for j in range(FV):
            tab_v[g, pl.ds(j * L, L)] = jnp.full((L,), NEG, jnp.float32)

    def point_body(k, carry):
        gvec = grp_v[pl.ds(k * L, L)]
        for m in range(L):
            g = gvec[m]
            i = k * L + m
            for j in range(FV):
                col = pl.ds(j * L, L)
                tab_v[g, col] = jnp.maximum(tab_v[g, col], feat_v[i, col])
        return carry

    lax.fori_loop(0, PTS // L, point_body, 0)

    vmin = grp_v[pl.ds(0, L)]
    vmax = vmin
    for k in range(1, PTS // L):
        w = grp_v[pl.ds(k * L, L)]
        vmin = jnp.minimum(vmin, w)
        vmax = jnp.maximum(vmax, w)
    cmin = jnp.min(vmin)
    cmax = jnp.max(vmax)
    uniform = cmin == cmax
    for j in range(FV):
        ids = lax.iota(jnp.int32, L) + j * L
        flag = jnp.where((ids == cmin) & uniform, 1.0, 0.0)
        full_v[pl.ds(j * L, L)] = flag

    pltpu.sync_copy(tab_v, hbm_tab.at[wid])
    pltpu.sync_copy(full_v, hbm_full.at[wid])
    plsc.subcore_barrier()

    w0 = c * NS + bl * TPB
    r = s % TPB
    cps = [pltpu.make_async_copy(
        hbm_tab.at[w0 + t, pl.ds(r * ROWS, ROWS), :], red_v.at[t], sem)
        for t in range(TPB)]
    cps.append(pltpu.make_async_copy(
        hbm_full.at[pl.ds(w0, TPB), :], fblk_v, sem))
    for cp in cps:
        cp.start()
    for cp in cps:
        cp.wait()

    for j in range(FV):
        col = pl.ds(j * L, L)
        fmin = fblk_v[0, col]
        for t in range(1, TPB):
            fmin = jnp.minimum(fmin, fblk_v[t, col])
        facc_v[col] = fmin
    fvec = facc_v[pl.ds(r * ROWS, L)]

    for rr in range(ROWS):
        fb = fvec[rr]
        floor = jnp.where(fb > 0.5, NEG, 0.0).astype(jnp.float32)
        for j in range(FV):
            col = pl.ds(j * L, L)
            m = jnp.maximum(red_v[0, rr, col], red_v[1, rr, col])
            for t in range(2, TPB):
                m = jnp.maximum(m, red_v[t, rr, col])
            acc_v[rr, col] = jnp.maximum(m, floor)

    pltpu.sync_copy(acc_v, gf_hbm.at[pl.ds(b * G + r * ROWS, ROWS), :])


@functools.cache
def _get_sc_call():
    return functools.partial(
        pl.kernel,
        out_type=jax.ShapeDtypeStruct((B * G, FD), jnp.float32),
        mesh=plsc.VectorSubcoreMesh(core_axis_name="c", subcore_axis_name="s",
                                    num_cores=NC, num_subcores=NS),
        compiler_params=pltpu.CompilerParams(needs_layout_passes=False),
        scratch_types=[
            pltpu.VMEM((PTS,), jnp.int32),
            pltpu.VMEM((PTS, FD), jnp.float32),
            pltpu.VMEM((G, FD), jnp.float32),
            pltpu.VMEM((G,), jnp.float32),
            pltpu.VMEM((TPB, ROWS, FD), jnp.float32),
            pltpu.VMEM((TPB, G), jnp.float32),
            pltpu.VMEM((G + L,), jnp.float32),
            pltpu.VMEM((ROWS, FD), jnp.float32),
            pltpu.SemaphoreType.DMA,
            pltpu.HBM((NW, G, FD), jnp.float32),
            pltpu.HBM((NW, G), jnp.float32),
        ],
    )(_sc_body)


def kernel(sphere, shape, features, w1, b1, g1, be1, w2, b2, g2, be2):
    xTs = sphere.transpose(2, 0, 1).reshape(3, BN)
    sTs = shape.transpose(2, 0, 1).reshape(3, BN)
    soft, wxyzT, wfoldT, groups, grpl = _tc_call(
        xTs, sTs, w1, b1, g1, be1, w2, b2, g2, be2)
    gf = _get_sc_call()(grpl, features)
    sf = _tc2_call(groups, gf)
    return (soft, wxyzT.transpose(1, 2, 0), groups, gf.reshape(B, G, FD),
            sf, wfoldT.transpose(1, 2, 0))


# speedup vs baseline: 1.6285x; 1.1404x over previous
import functools

import jax
import jax.numpy as jnp
from jax import lax
from jax.experimental import pallas as pl
from jax.experimental.pallas import tpu as pltpu
from jax.experimental.pallas import tpu_sc as plsc

B, N, FD, G = 4, 2048, 64, 64
H = 128
BN = B * N
NC, NS, L = 2, 16, 16
NW = NC * NS
PTS = BN // NW
TPB = NW // B
ROWS = G // TPB
FV = FD // L
NEG = -3.0e38


def _eye(n):
    return (lax.broadcasted_iota(jnp.int32, (n, n), 0) ==
            lax.broadcasted_iota(jnp.int32, (n, n), 1)).astype(jnp.float32)


def _col(v_ref, eye):
    return lax.dot_general(eye, v_ref[...][None, :], (((1,), (1,)), ((), ())),
                           preferred_element_type=jnp.float32)


def _tc_body(xT_ref, sT_ref, fT_ref, w1_ref, b1_ref, g1_ref, be1_ref,
             w2_ref, b2_ref, g2_ref, be2_ref,
             soft_ref, wxyz_ref, wfold_ref, grp_ref, grpl_ref, featn_ref):
    dn = (((1,), (1,)), ((), ()))
    eyeH, eyeG = _eye(H), _eye(G)
    xT = xT_ref[...]
    h = jnp.dot(w1_ref[...], xT, preferred_element_type=jnp.float32)
    h = h + _col(b1_ref, eyeH)
    mean = jnp.mean(h, axis=1, keepdims=True)
    var = jnp.mean((h - mean) ** 2, axis=1, keepdims=True)
    h = (_col(g1_ref, eyeH) * (h - mean) * lax.rsqrt(var + 1e-5)
         + _col(be1_ref, eyeH))
    h = jnp.maximum(h, 0.0)
    s = jnp.dot(w2_ref[...], h, preferred_element_type=jnp.float32)
    s = s + _col(b2_ref, eyeG)
    mean2 = jnp.mean(s, axis=1, keepdims=True)
    var2 = jnp.mean((s - mean2) ** 2, axis=1, keepdims=True)
    s = (_col(g2_ref, eyeG) * (s - mean2) * lax.rsqrt(var2 + 1e-5)
         + _col(be2_ref, eyeG))
    for b in range(B):
        sb = s[:, b * N:(b + 1) * N]
        m = jnp.max(sb, axis=1, keepdims=True)
        e = jnp.exp(sb - m)
        soft_b = e / jnp.sum(e, axis=1, keepdims=True)
        soft_ref[b] = soft_b
        wxyz_ref[:, b, :] = lax.dot_general(
            xT[:, b * N:(b + 1) * N], soft_b, dn,
            preferred_element_type=jnp.float32)
        wfold_ref[:, b, :] = lax.dot_general(
            sT_ref[:, b * N:(b + 1) * N], soft_b, dn,
            preferred_element_type=jnp.float32)
        mcol = jnp.max(sb, axis=0, keepdims=True)
        ids = lax.broadcasted_iota(jnp.int32, (G, N), 0)
        grp_b = jnp.min(jnp.where(sb == mcol, ids, G), axis=0)
        grp_ref[b] = grp_b
        grpl_ref[pl.ds(b * N, N)] = grp_b
        featn_ref[pl.ds(b * N, N), :] = lax.dot_general(
            fT_ref[b], eyeG, (((0,), (0,)), ((), ())),
            preferred_element_type=jnp.float32)


_tc_call = pl.pallas_call(
    _tc_body,
    out_shape=(
        jax.ShapeDtypeStruct((B, G, N), jnp.float32),
        jax.ShapeDtypeStruct((3, B, G), jnp.float32),
        jax.ShapeDtypeStruct((3, B, G), jnp.float32),
        jax.ShapeDtypeStruct((B, N), jnp.int32),
        jax.ShapeDtypeStruct((BN,), jnp.int32),
        jax.ShapeDtypeStruct((BN, FD), jnp.float32),
    ),
)


def _tc2_body(grp_ref, gf_ref, sf_ref):
    for b in range(B):
        grp_row = grp_ref[b][None, :]
        ohT = (lax.broadcasted_iota(jnp.int32, (G, N), 0) == grp_row)
        sf_ref[b] = lax.dot_general(
            gf_ref[b * G:(b + 1) * G, :], ohT.astype(jnp.float32),
            (((0,), (0,)), ((), ())), preferred_element_type=jnp.float32)


_tc2_call = pl.pallas_call(
    _tc2_body,
    out_shape=jax.ShapeDtypeStruct((B, FD, N), jnp.float32),
)


def _sc_body(grp_hbm, feat_hbm, gf_hbm,
             grp_v, feat_v, tab_v, full_v, red_v, fblk_v, facc_v, acc_v,
             sem, hbm_tab, hbm_full):
    c = lax.axis_index("c")
    s = lax.axis_index("s")
    bl = s // TPB
    b = c * 2 + bl
    chunk = s % TPB
    pt0 = b * N + chunk * PTS
    wid = c * NS + s

    pltpu.sync_copy(grp_hbm.at[pl.ds(pt0, PTS)], grp_v)
    pltpu.sync_copy(feat_hbm.at[pl.ds(pt0, PTS), :], feat_v)

    for g in range(G):
        for j in range(FV):
            tab_v[g, pl.ds(j * L, L)] = jnp.full((L,), NEG, jnp.float32)

    def point_body(k, carry):
        gvec = grp_v[pl.ds(k * L, L)]
        for m in range(L):
            g = gvec[m]
            i = k * L + m
            for j in range(FV):
                col = pl.ds(j * L, L)
                tab_v[g, col] = jnp.maximum(tab_v[g, col], feat_v[i, col])
        return carry

    lax.fori_loop(0, PTS // L, point_body, 0)

    vmin = grp_v[pl.ds(0, L)]
    vmax = vmin
    for k in range(1, PTS // L):
        w = grp_v[pl.ds(k * L, L)]
        vmin = jnp.minimum(vmin, w)
        vmax = jnp.maximum(vmax, w)
    cmin = jnp.min(vmin)
    cmax = jnp.max(vmax)
    uniform = cmin == cmax
    for j in range(FV):
        ids = lax.iota(jnp.int32, L) + j * L
        flag = jnp.where((ids == cmin) & uniform, 1.0, 0.0)
        full_v[pl.ds(j * L, L)] = flag

    pltpu.sync_copy(tab_v, hbm_tab.at[wid])
    pltpu.sync_copy(full_v, hbm_full.at[wid])
    plsc.subcore_barrier()

    w0 = c * NS + bl * TPB
    r = s % TPB
    cps = [pltpu.make_async_copy(
        hbm_tab.at[w0 + t, pl.ds(r * ROWS, ROWS), :], red_v.at[t], sem)
        for t in range(TPB)]
    cps.append(pltpu.make_async_copy(
        hbm_full.at[pl.ds(w0, TPB), :], fblk_v, sem))
    for cp in cps:
        cp.start()
    for cp in cps:
        cp.wait()

    for j in range(FV):
        col = pl.ds(j * L, L)
        fmin = fblk_v[0, col]
        for t in range(1, TPB):
            fmin = jnp.minimum(fmin, fblk_v[t, col])
        facc_v[col] = fmin
    fvec = facc_v[pl.ds(r * ROWS, L)]

    for rr in range(ROWS):
        fb = fvec[rr]
        floor = jnp.where(fb > 0.5, NEG, 0.0).astype(jnp.float32)
        for j in range(FV):
            col = pl.ds(j * L, L)
            m = jnp.maximum(red_v[0, rr, col], red_v[1, rr, col])
            for t in range(2, TPB):
                m = jnp.maximum(m, red_v[t, rr, col])
            acc_v[rr, col] = jnp.maximum(m, floor)

    pltpu.sync_copy(acc_v, gf_hbm.at[pl.ds(b * G + r * ROWS, ROWS), :])


@functools.cache
def _get_sc_call():
    return functools.partial(
        pl.kernel,
        out_type=jax.ShapeDtypeStruct((B * G, FD), jnp.float32),
        mesh=plsc.VectorSubcoreMesh(core_axis_name="c", subcore_axis_name="s",
                                    num_cores=NC, num_subcores=NS),
        compiler_params=pltpu.CompilerParams(needs_layout_passes=False),
        scratch_types=[
            pltpu.VMEM((PTS,), jnp.int32),
            pltpu.VMEM((PTS, FD), jnp.float32),
            pltpu.VMEM((G, FD), jnp.float32),
            pltpu.VMEM((G,), jnp.float32),
            pltpu.VMEM((TPB, ROWS, FD), jnp.float32),
            pltpu.VMEM((TPB, G), jnp.float32),
            pltpu.VMEM((G + L,), jnp.float32),
            pltpu.VMEM((ROWS, FD), jnp.float32),
            pltpu.SemaphoreType.DMA,
            pltpu.HBM((NW, G, FD), jnp.float32),
            pltpu.HBM((NW, G), jnp.float32),
        ],
    )(_sc_body)


def kernel(sphere, shape, features, w1, b1, g1, be1, w2, b2, g2, be2):
    xTs = sphere.transpose(2, 0, 1).reshape(3, BN)
    sTs = shape.transpose(2, 0, 1).reshape(3, BN)
    featT = features.transpose(0, 2, 1)
    soft, wxyzT, wfoldT, groups, grpl, featn = _tc_call(
        xTs, sTs, featT, w1, b1, g1, be1, w2, b2, g2, be2)
    gf = _get_sc_call()(grpl, featn)
    sfT = _tc2_call(groups, gf)
    return (soft, wxyzT.transpose(1, 2, 0), groups, gf.reshape(B, G, FD),
            sfT.transpose(0, 2, 1), wfoldT.transpose(1, 2, 0))


# speedup vs baseline: 1.6530x; 1.0150x over previous
import functools

import jax
import jax.numpy as jnp
from jax import lax
from jax.experimental import pallas as pl
from jax.experimental.pallas import tpu as pltpu
from jax.experimental.pallas import tpu_sc as plsc

B, N, FD, G = 4, 2048, 64, 64
H = 128
BN = B * N
NC, NS, L = 2, 16, 16
NW = NC * NS
PTS = BN // NW
TPB = NW // B
ROWS = G // TPB
FV = FD // L
NEG = -3.0e38


def _eye(n):
    return (lax.broadcasted_iota(jnp.int32, (n, n), 0) ==
            lax.broadcasted_iota(jnp.int32, (n, n), 1)).astype(jnp.float32)


def _col(v_ref, eye):
    return lax.dot_general(eye, v_ref[...][None, :], (((1,), (1,)), ((), ())),
                           preferred_element_type=jnp.float32)


def _tc_body(xT_ref, sT_ref, fT_ref, w1_ref, b1_ref, g1_ref, be1_ref,
             w2_ref, b2_ref, g2_ref, be2_ref,
             soft_ref, wxyz_ref, wfold_ref, grp_ref, grpl_ref, featn_ref):
    dn = (((1,), (1,)), ((), ()))
    eyeH, eyeG = _eye(H), _eye(G)
    xT = xT_ref[...]
    h = jnp.dot(w1_ref[...], xT, preferred_element_type=jnp.float32)
    h = h + _col(b1_ref, eyeH)
    mean = jnp.mean(h, axis=1, keepdims=True)
    var = jnp.mean((h - mean) ** 2, axis=1, keepdims=True)
    h = (_col(g1_ref, eyeH) * (h - mean) * lax.rsqrt(var + 1e-5)
         + _col(be1_ref, eyeH))
    h = jnp.maximum(h, 0.0)
    s = jnp.dot(w2_ref[...], h, preferred_element_type=jnp.float32)
    s = s + _col(b2_ref, eyeG)
    mean2 = jnp.mean(s, axis=1, keepdims=True)
    var2 = jnp.mean((s - mean2) ** 2, axis=1, keepdims=True)
    s = (_col(g2_ref, eyeG) * (s - mean2) * lax.rsqrt(var2 + 1e-5)
         + _col(be2_ref, eyeG))
    for b in range(B):
        sb = s[:, b * N:(b + 1) * N]
        m = jnp.max(sb, axis=1, keepdims=True)
        e = jnp.exp(sb - m)
        soft_b = e / jnp.sum(e, axis=1, keepdims=True)
        soft_ref[b] = soft_b
        wxyz_ref[:, b, :] = lax.dot_general(
            xT[:, b * N:(b + 1) * N], soft_b, dn,
            preferred_element_type=jnp.float32)
        wfold_ref[:, b, :] = lax.dot_general(
            sT_ref[:, b * N:(b + 1) * N], soft_b, dn,
            preferred_element_type=jnp.float32)
        mcol = jnp.max(sb, axis=0, keepdims=True)
        ids = lax.broadcasted_iota(jnp.int32, (G, N), 0)
        grp_b = jnp.min(jnp.where(sb == mcol, ids, G), axis=0)
        grp_ref[b] = grp_b
        grpl_ref[pl.ds(b * N, N)] = grp_b
        featn_ref[pl.ds(b * N, N), :] = lax.dot_general(
            fT_ref[b], eyeG, (((0,), (0,)), ((), ())),
            preferred_element_type=jnp.float32)


_tc_call = pl.pallas_call(
    _tc_body,
    out_shape=(
        jax.ShapeDtypeStruct((B, G, N), jnp.float32),
        jax.ShapeDtypeStruct((3, B, G), jnp.float32),
        jax.ShapeDtypeStruct((3, B, G), jnp.float32),
        jax.ShapeDtypeStruct((B, N), jnp.int32),
        jax.ShapeDtypeStruct((BN,), jnp.int32),
        jax.ShapeDtypeStruct((BN, FD), jnp.float32),
    ),
)


def _tc2_body(grp_ref, gf_ref, sf_ref):
    for b in range(B):
        grp_row = grp_ref[b][None, :]
        ohT = (lax.broadcasted_iota(jnp.int32, (G, N), 0) == grp_row)
        sf_ref[b] = lax.dot_general(
            gf_ref[b * G:(b + 1) * G, :], ohT.astype(jnp.float32),
            (((0,), (0,)), ((), ())), preferred_element_type=jnp.float32)


_tc2_call = pl.pallas_call(
    _tc2_body,
    out_shape=jax.ShapeDtypeStruct((B, FD, N), jnp.float32),
)


def _sc_body(grp_hbm, feat_hbm, gf_hbm,
             grp_v, feat_v, tab_v, full_v, red_v, fblk_v, facc_v, acc_v,
             sem, hbm_tab, hbm_full):
    c = lax.axis_index("c")
    s = lax.axis_index("s")
    bl = s // TPB
    b = c * 2 + bl
    chunk = s % TPB
    pt0 = b * N + chunk * PTS
    wid = c * NS + s

    cin0 = pltpu.make_async_copy(grp_hbm.at[pl.ds(pt0, PTS)], grp_v, sem)
    cin1 = pltpu.make_async_copy(feat_hbm.at[pl.ds(pt0, PTS), :], feat_v, sem)
    cin0.start()
    cin1.start()

    for g in range(G):
        for j in range(FV):
            tab_v[g, pl.ds(j * L, L)] = jnp.full((L,), NEG, jnp.float32)
    cin0.wait()
    cin1.wait()

    def point_body(k, carry):
        gvec = grp_v[pl.ds(k * L, L)]
        for m in range(L):
            g = gvec[m]
            i = k * L + m
            for j in range(FV):
                col = pl.ds(j * L, L)
                tab_v[g, col] = jnp.maximum(tab_v[g, col], feat_v[i, col])
        return carry

    lax.fori_loop(0, PTS // L, point_body, 0)

    vmin = grp_v[pl.ds(0, L)]
    vmax = vmin
    for k in range(1, PTS // L):
        w = grp_v[pl.ds(k * L, L)]
        vmin = jnp.minimum(vmin, w)
        vmax = jnp.maximum(vmax, w)
    cmin = jnp.min(vmin)
    cmax = jnp.max(vmax)
    uniform = cmin == cmax
    for j in range(FV):
        ids = lax.iota(jnp.int32, L) + j * L
        flag = jnp.where((ids == cmin) & uniform, 1.0, 0.0)
        full_v[pl.ds(j * L, L)] = flag

    st0 = pltpu.make_async_copy(tab_v, hbm_tab.at[wid], sem)
    st1 = pltpu.make_async_copy(full_v, hbm_full.at[wid], sem)
    st0.start()
    st1.start()
    st0.wait()
    st1.wait()
    plsc.subcore_barrier()

    w0 = c * NS + bl * TPB
    r = s % TPB
    cps = [pltpu.make_async_copy(
        hbm_tab.at[w0 + t, pl.ds(r * ROWS, ROWS), :], red_v.at[t], sem)
        for t in range(TPB)]
    cps.append(pltpu.make_async_copy(
        hbm_full.at[pl.ds(w0, TPB), :], fblk_v, sem))
    for cp in cps:
        cp.start()
    for cp in cps:
        cp.wait()

    for j in range(FV):
        col = pl.ds(j * L, L)
        fmin = fblk_v[0, col]
        for t in range(1, TPB):
            fmin = jnp.minimum(fmin, fblk_v[t, col])
        facc_v[col] = fmin
    fvec = facc_v[pl.ds(r * ROWS, L)]

    for rr in range(ROWS):
        fb = fvec[rr]
        floor = jnp.where(fb > 0.5, NEG, 0.0).astype(jnp.float32)
        for j in range(FV):
            col = pl.ds(j * L, L)
            m = jnp.maximum(red_v[0, rr, col], red_v[1, rr, col])
            for t in range(2, TPB):
                m = jnp.maximum(m, red_v[t, rr, col])
            acc_v[rr, col] = jnp.maximum(m, floor)

    pltpu.sync_copy(acc_v, gf_hbm.at[pl.ds(b * G + r * ROWS, ROWS), :])


@functools.cache
def _get_sc_call():
    return functools.partial(
        pl.kernel,
        out_type=jax.ShapeDtypeStruct((B * G, FD), jnp.float32),
        mesh=plsc.VectorSubcoreMesh(core_axis_name="c", subcore_axis_name="s",
                                    num_cores=NC, num_subcores=NS),
        compiler_params=pltpu.CompilerParams(needs_layout_passes=False),
        scratch_types=[
            pltpu.VMEM((PTS,), jnp.int32),
            pltpu.VMEM((PTS, FD), jnp.float32),
            pltpu.VMEM((G, FD), jnp.float32),
            pltpu.VMEM((G,), jnp.float32),
            pltpu.VMEM((TPB, ROWS, FD), jnp.float32),
            pltpu.VMEM((TPB, G), jnp.float32),
            pltpu.VMEM((G + L,), jnp.float32),
            pltpu.VMEM((ROWS, FD), jnp.float32),
            pltpu.SemaphoreType.DMA,
            pltpu.HBM((NW, G, FD), jnp.float32),
            pltpu.HBM((NW, G), jnp.float32),
        ],
    )(_sc_body)


def kernel(sphere, shape, features, w1, b1, g1, be1, w2, b2, g2, be2):
    xTs = sphere.transpose(2, 0, 1).reshape(3, BN)
    sTs = shape.transpose(2, 0, 1).reshape(3, BN)
    featT = features.transpose(0, 2, 1)
    soft, wxyzT, wfoldT, groups, grpl, featn = _tc_call(
        xTs, sTs, featT, w1, b1, g1, be1, w2, b2, g2, be2)
    gf = _get_sc_call()(grpl, featn)
    sfT = _tc2_call(groups, gf)
    return (soft, wxyzT.transpose(1, 2, 0), groups, gf.reshape(B, G, FD),
            sfT.transpose(0, 2, 1), wfoldT.transpose(1, 2, 0))


# speedup vs baseline: 1.7234x; 1.0426x over previous
import functools

import jax
import jax.numpy as jnp
from jax import lax
from jax.experimental import pallas as pl
from jax.experimental.pallas import tpu as pltpu
from jax.experimental.pallas import tpu_sc as plsc

B, N, FD, G = 4, 2048, 64, 64
H = 128
BN = B * N
NC, NS, L = 2, 16, 16
NW = NC * NS
PTS = BN // NW
TPB = NW // B
ROWS = G // TPB
FV = FD // L
NEG = -3.0e38


def _eye(n):
    return (lax.broadcasted_iota(jnp.int32, (n, n), 0) ==
            lax.broadcasted_iota(jnp.int32, (n, n), 1)).astype(jnp.float32)


def _col(v_ref, eye):
    return lax.dot_general(eye, v_ref[...][None, :], (((1,), (1,)), ((), ())),
                           preferred_element_type=jnp.float32)


def _tc_body(xs_ref, fT_ref, w1t_ref, b1_ref, g1_ref, be1_ref,
             w2_ref, b2_ref, g2_ref, be2_ref,
             soft_ref, wxyz_ref, wfold_ref, grp_ref, grpl_ref, featn_ref):
    dn = (((1,), (1,)), ((), ()))
    d0 = (((0,), (0,)), ((), ()))
    eyeH, eyeG = _eye(H), _eye(G)
    xs = xs_ref[...]
    xT = xs[:3]
    sT = xs[3:]
    h = lax.dot_general(w1t_ref[...], xT, d0,
                        preferred_element_type=jnp.float32)
    h = h + _col(b1_ref, eyeH)
    mean = jnp.mean(h, axis=1, keepdims=True)
    var = jnp.mean((h - mean) ** 2, axis=1, keepdims=True)
    h = (_col(g1_ref, eyeH) * (h - mean) * lax.rsqrt(var + 1e-5)
         + _col(be1_ref, eyeH))
    h = jnp.maximum(h, 0.0)
    s = jnp.dot(w2_ref[...], h, preferred_element_type=jnp.float32)
    s = s + _col(b2_ref, eyeG)
    mean2 = jnp.mean(s, axis=1, keepdims=True)
    var2 = jnp.mean((s - mean2) ** 2, axis=1, keepdims=True)
    s = (_col(g2_ref, eyeG) * (s - mean2) * lax.rsqrt(var2 + 1e-5)
         + _col(be2_ref, eyeG))
    for b in range(B):
        sb = s[:, b * N:(b + 1) * N]
        m = jnp.max(sb, axis=1, keepdims=True)
        e = jnp.exp(sb - m)
        soft_b = e / jnp.sum(e, axis=1, keepdims=True)
        soft_ref[b] = soft_b
        wxyz_ref[:, b, :] = lax.dot_general(
            xT[:, b * N:(b + 1) * N], soft_b, dn,
            preferred_element_type=jnp.float32)
        wfold_ref[:, b, :] = lax.dot_general(
            sT[:, b * N:(b + 1) * N], soft_b, dn,
            preferred_element_type=jnp.float32)
        mcol = jnp.max(sb, axis=0, keepdims=True)
        ids = lax.broadcasted_iota(jnp.int32, (G, N), 0)
        grp_b = jnp.min(jnp.where(sb == mcol, ids, G), axis=0)
        grp_ref[b] = grp_b
        grpl_ref[pl.ds(b * N, N)] = grp_b
        featn_ref[pl.ds(b * N, N), :] = lax.dot_general(
            fT_ref[b], eyeG, (((0,), (0,)), ((), ())),
            preferred_element_type=jnp.float32)


_tc_call = pl.pallas_call(
    _tc_body,
    out_shape=(
        jax.ShapeDtypeStruct((B, G, N), jnp.float32),
        jax.ShapeDtypeStruct((3, B, G), jnp.float32),
        jax.ShapeDtypeStruct((3, B, G), jnp.float32),
        jax.ShapeDtypeStruct((B, N), jnp.int32),
        jax.ShapeDtypeStruct((BN,), jnp.int32),
        jax.ShapeDtypeStruct((BN, FD), jnp.float32),
    ),
)


def _tc2_body(grp_ref, gf_ref, sf_ref):
    for b in range(B):
        grp_row = grp_ref[b][None, :]
        ohT = (lax.broadcasted_iota(jnp.int32, (G, N), 0) == grp_row)
        sf_ref[b] = lax.dot_general(
            gf_ref[b * G:(b + 1) * G, :], ohT.astype(jnp.float32),
            (((0,), (0,)), ((), ())), preferred_element_type=jnp.float32)


_tc2_call = pl.pallas_call(
    _tc2_body,
    out_shape=jax.ShapeDtypeStruct((B, FD, N), jnp.float32),
)


def _sc_body(grp_hbm, feat_hbm, gf_hbm,
             grp_v, feat_v, tab_v, full_v, red_v, fblk_v, facc_v, acc_v,
             sem, hbm_tab, hbm_full):
    c = lax.axis_index("c")
    s = lax.axis_index("s")
    bl = s // TPB
    b = c * 2 + bl
    chunk = s % TPB
    pt0 = b * N + chunk * PTS
    wid = c * NS + s

    cin0 = pltpu.make_async_copy(grp_hbm.at[pl.ds(pt0, PTS)], grp_v, sem)
    cin1 = pltpu.make_async_copy(feat_hbm.at[pl.ds(pt0, PTS), :], feat_v, sem)
    cin0.start()
    cin1.start()

    for g in range(G):
        for j in range(FV):
            tab_v[g, pl.ds(j * L, L)] = jnp.full((L,), NEG, jnp.float32)
    cin0.wait()
    cin1.wait()

    def point_body(k, carry):
        gvec = grp_v[pl.ds(k * L, L)]
        for m in range(L):
            g = gvec[m]
            i = k * L + m
            for j in range(FV):
                col = pl.ds(j * L, L)
                tab_v[g, col] = jnp.maximum(tab_v[g, col], feat_v[i, col])
        return carry

    lax.fori_loop(0, PTS // L, point_body, 0)

    vmin = grp_v[pl.ds(0, L)]
    vmax = vmin
    for k in range(1, PTS // L):
        w = grp_v[pl.ds(k * L, L)]
        vmin = jnp.minimum(vmin, w)
        vmax = jnp.maximum(vmax, w)
    cmin = jnp.min(vmin)
    cmax = jnp.max(vmax)
    uniform = cmin == cmax
    for j in range(FV):
        ids = lax.iota(jnp.int32, L) + j * L
        flag = jnp.where((ids == cmin) & uniform, 1.0, 0.0)
        full_v[pl.ds(j * L, L)] = flag

    st0 = pltpu.make_async_copy(tab_v, hbm_tab.at[wid], sem)
    st1 = pltpu.make_async_copy(full_v, hbm_full.at[wid], sem)
    st0.start()
    st1.start()
    st0.wait()
    st1.wait()
    plsc.subcore_barrier()

    w0 = c * NS + bl * TPB
    r = s % TPB
    cps = [pltpu.make_async_copy(
        hbm_tab.at[w0 + t, pl.ds(r * ROWS, ROWS), :], red_v.at[t], sem)
        for t in range(TPB)]
    cps.append(pltpu.make_async_copy(
        hbm_full.at[pl.ds(w0, TPB), :], fblk_v, sem))
    for cp in cps:
        cp.start()
    for cp in cps:
        cp.wait()

    for j in range(FV):
        col = pl.ds(j * L, L)
        fmin = fblk_v[0, col]
        for t in range(1, TPB):
            fmin = jnp.minimum(fmin, fblk_v[t, col])
        facc_v[col] = fmin
    fvec = facc_v[pl.ds(r * ROWS, L)]

    for rr in range(ROWS):
        fb = fvec[rr]
        floor = jnp.where(fb > 0.5, NEG, 0.0).astype(jnp.float32)
        for j in range(FV):
            col = pl.ds(j * L, L)
            m = jnp.maximum(red_v[0, rr, col], red_v[1, rr, col])
            for t in range(2, TPB):
                m = jnp.maximum(m, red_v[t, rr, col])
            acc_v[rr, col] = jnp.maximum(m, floor)

    pltpu.sync_copy(acc_v, gf_hbm.at[pl.ds(b * G + r * ROWS, ROWS), :])


@functools.cache
def _get_sc_call():
    return functools.partial(
        pl.kernel,
        out_type=jax.ShapeDtypeStruct((B * G, FD), jnp.float32),
        mesh=plsc.VectorSubcoreMesh(core_axis_name="c", subcore_axis_name="s",
                                    num_cores=NC, num_subcores=NS),
        compiler_params=pltpu.CompilerParams(needs_layout_passes=False),
        scratch_types=[
            pltpu.VMEM((PTS,), jnp.int32),
            pltpu.VMEM((PTS, FD), jnp.float32),
            pltpu.VMEM((G, FD), jnp.float32),
            pltpu.VMEM((G,), jnp.float32),
            pltpu.VMEM((TPB, ROWS, FD), jnp.float32),
            pltpu.VMEM((TPB, G), jnp.float32),
            pltpu.VMEM((G + L,), jnp.float32),
            pltpu.VMEM((ROWS, FD), jnp.float32),
            pltpu.SemaphoreType.DMA,
            pltpu.HBM((NW, G, FD), jnp.float32),
            pltpu.HBM((NW, G), jnp.float32),
        ],
    )(_sc_body)


def kernel(sphere, shape, features, w1, b1, g1, be1, w2, b2, g2, be2):
    featT = features.transpose(0, 2, 1)
    xs = jnp.concatenate([sphere.transpose(2, 0, 1).reshape(3, BN),
                          shape.transpose(2, 0, 1).reshape(3, BN)], axis=0)
    soft, wxyzT, wfoldT, groups, grpl, featn = _tc_call(
        xs, featT, w1.T, b1, g1, be1, w2, b2, g2, be2)
    gf = _get_sc_call()(grpl, featn)
    sfT = _tc2_call(groups, gf)
    return (soft, wxyzT.transpose(1, 2, 0), groups, gf.reshape(B, G, FD),
            sfT.transpose(0, 2, 1), wfoldT.transpose(1, 2, 0))
